# baseline (speedup 1.0000x reference)
import functools

import jax
import jax.numpy as jnp
from jax import lax
from jax.experimental import pallas as pl
from jax.experimental.pallas import tpu as pltpu
from jax.experimental.pallas import tpu_sc as plsc

R = 128
N = 32768
L = 16
NC = 2
NS = 16
NW = NC * NS
ROWS_PER_W = R // NW
VECS = N // L

_INT_MAX = 2**31 - 1


@functools.partial(
    pl.kernel,
    mesh=plsc.VectorSubcoreMesh(core_axis_name="c", subcore_axis_name="s"),
    out_type=jax.ShapeDtypeStruct((NW, L), jnp.int32),
    scratch_types=[
        pltpu.VMEM((2, N), jnp.float32),
        pltpu.VMEM((L,), jnp.int32),
        pltpu.VMEM((L,), jnp.float32),
        pltpu.VMEM((L,), jnp.int32),
    ],
    compiler_params=pltpu.CompilerParams(needs_layout_passes=False),
)
def _argmin_sc(x_hbm, out_hbm, buf, outbuf, redv, redi):
    wid = lax.axis_index("s") * NC + lax.axis_index("c")
    base_iota = lax.iota(jnp.int32, L)

    results = jnp.zeros((L,), dtype=jnp.int32)
    for r in range(ROWS_PER_W):
        row = wid * ROWS_PER_W + r
        slot = r % 2
        pltpu.sync_copy(x_hbm.at[row], buf.at[slot])

        def body(i, carry):
            mv, mi, iv = carry
            v = buf[slot, pl.ds(i * L, L)]
            m = v < mv
            mv = jnp.where(m, v, mv)
            mi = jnp.where(m, iv, mi)
            return mv, mi, iv + L

        mv0 = jnp.full((L,), jnp.inf, dtype=jnp.float32)
        mi0 = jnp.zeros((L,), dtype=jnp.int32)
        mv, mi, _ = lax.fori_loop(0, VECS, body, (mv0, mi0, base_iota))

        for sh in (8, 4, 2, 1):
            redv[...] = mv
            redi[...] = mi
            perm = base_iota ^ sh
            ov = plsc.load_gather(redv, [perm])
            oi = plsc.load_gather(redi, [perm])
            take = (ov < mv) | ((ov == mv) & (oi < mi))
            mv = jnp.where(take, ov, mv)
            mi = jnp.where(take, oi, mi)
        results = jnp.where(base_iota == r, mi, results)

    outbuf[...] = results
    pltpu.sync_copy(outbuf, out_hbm.at[wid])


def kernel(x):
    staged = _argmin_sc(x)
    return staged[:, :ROWS_PER_W].reshape(R, 1)


# speedup vs baseline: 1.7163x; 1.7163x over previous
import functools

import jax
import jax.numpy as jnp
from jax import lax
from jax.experimental import pallas as pl
from jax.experimental.pallas import tpu as pltpu
from jax.experimental.pallas import tpu_sc as plsc

R = 128
N = 32768
L = 16
NC = 2
NS = 16
NW = NC * NS
ROWS_PER_W = R // NW
ACCS = 8
STEPS = N // (ACCS * L)

_INT_MAX = 2**31 - 1


@functools.partial(
    pl.kernel,
    mesh=plsc.VectorSubcoreMesh(core_axis_name="c", subcore_axis_name="s"),
    out_type=jax.ShapeDtypeStruct((NW, L), jnp.int32),
    scratch_types=[
        pltpu.VMEM((2, N), jnp.float32),
        pltpu.VMEM((L,), jnp.int32),
        pltpu.VMEM((L,), jnp.float32),
        pltpu.VMEM((L,), jnp.int32),
        pltpu.SemaphoreType.DMA,
        pltpu.SemaphoreType.DMA,
    ],
    compiler_params=pltpu.CompilerParams(needs_layout_passes=False),
)
def _argmin_sc(x_hbm, out_hbm, buf, outbuf, redv, redi, sem0, sem1):
    wid = lax.axis_index("s") * NC + lax.axis_index("c")
    base_iota = lax.iota(jnp.int32, L)
    sems = (sem0, sem1)

    row0 = wid * ROWS_PER_W
    pending = pltpu.async_copy(x_hbm.at[row0], buf.at[0], sems[0])

    results = jnp.zeros((L,), dtype=jnp.int32)
    for r in range(ROWS_PER_W):
        slot = r % 2
        pending.wait()
        if r + 1 < ROWS_PER_W:
            nxt = (r + 1) % 2
            pending = pltpu.async_copy(
                x_hbm.at[row0 + r + 1], buf.at[nxt], sems[nxt]
            )

        mv0 = tuple(
            jnp.full((L,), jnp.inf, dtype=jnp.float32) for _ in range(ACCS)
        )
        mt0 = tuple(jnp.zeros((L,), dtype=jnp.int32) for _ in range(ACCS))

        @plsc.parallel_loop(0, STEPS, 1, unroll=4, carry=(mv0, mt0))
        def _scan(t, carry):
            mvs, mts = carry
            tb = jnp.full((L,), t, dtype=jnp.int32)
            new_mvs = []
            new_mts = []
            for k in range(ACCS):
                v = buf[slot, pl.ds(t * (ACCS * L) + k * L, L)]
                m = v < mvs[k]
                new_mvs.append(jnp.where(m, v, mvs[k]))
                new_mts.append(jnp.where(m, tb, mts[k]))
            return tuple(new_mvs), tuple(new_mts)

        mvs, mts = _scan
        mv = mvs[0]
        mi = mts[0] * (ACCS * L) + base_iota
        for k in range(1, ACCS):
            fi = mts[k] * (ACCS * L) + (k * L + base_iota)
            take = (mvs[k] < mv) | ((mvs[k] == mv) & (fi < mi))
            mv = jnp.where(take, mvs[k], mv)
            mi = jnp.where(take, fi, mi)

        for sh in (8, 4, 2, 1):
            redv[...] = mv
            redi[...] = mi
            perm = base_iota ^ sh
            ov = plsc.load_gather(redv, [perm])
            oi = plsc.load_gather(redi, [perm])
            take = (ov < mv) | ((ov == mv) & (oi < mi))
            mv = jnp.where(take, ov, mv)
            mi = jnp.where(take, oi, mi)
        results = jnp.where(base_iota == r, mi, results)

    outbuf[...] = results
    pltpu.sync_copy(outbuf, out_hbm.at[wid])


def kernel(x):
    staged = _argmin_sc(x)
    return staged[:, :ROWS_PER_W].reshape(R, 1)
